# Initial kernel scaffold; baseline (speedup 1.0000x reference)
import jax
import jax.numpy as jnp
from jax.experimental import pallas as pl


def kernel(x, edge_index, batchsize, edge_weight, W1, b1, W2, b2, W3, b3, gamma, beta):
    raise NotImplementedError("write your pallas kernel here")


# speedup vs baseline: 16.5314x; 16.5314x over previous
import functools

import jax
import jax.numpy as jnp
from jax import lax
from jax.experimental import pallas as pl
from jax.experimental.pallas import tpu as pltpu
from jax.experimental.pallas import tpu_sc as plsc

NC = 2
NS = 16
NW = NC * NS
CHUNK = 128
DH = 64

_MESH = plsc.VectorSubcoreMesh(core_axis_name="c", subcore_axis_name="s")


def _deg_body(npad, dst_hbm, w_hbm, out_hbm, dstb, wb, zb, shared):
    cid = lax.axis_index("c")
    sid = lax.axis_index("s")
    wid = sid * NC + cid
    cpt = dst_hbm.shape[0] // NW
    zslice = npad // NS

    def zf(i, carry):
        zb[pl.ds(i * 16, 16)] = jnp.zeros((16,), jnp.float32)
        return carry
    lax.fori_loop(0, zslice // 16, zf, 0)
    pltpu.sync_copy(zb, shared.at[pl.ds(sid * zslice, zslice)])
    plsc.subcore_barrier()

    pltpu.sync_copy(dst_hbm.at[pl.ds(wid * cpt, cpt)], dstb)
    pltpu.sync_copy(w_hbm.at[pl.ds(wid * cpt, cpt)], wb)

    def body(g, carry):
        pltpu.sync_copy(wb.at[g], shared.at[dstb.at[g]], add=True)
        return carry
    lax.fori_loop(0, cpt, body, 0)
    plsc.subcore_barrier()
    sl = pl.ds(sid * zslice, zslice)
    pltpu.sync_copy(shared.at[sl], out_hbm.at[cid, sl])


def _deg_call(dst_p, w_p, npad):
    nchunks = dst_p.shape[0]
    cpt = nchunks // NW
    return pl.kernel(
        functools.partial(_deg_body, npad),
        out_type=jax.ShapeDtypeStruct((NC, npad), jnp.float32),
        mesh=_MESH,
        scratch_types=[
            pltpu.VMEM((cpt, CHUNK), jnp.int32),
            pltpu.VMEM((cpt, CHUNK), jnp.float32),
            pltpu.VMEM((npad // NS,), jnp.float32),
            pltpu.VMEM_SHARED((npad,), jnp.float32),
        ],
    )(dst_p, w_p)


def _spmm_body(n, npad, u_hbm, src_hbm, dst_hbm, w_hbm, out_hbm,
               srcb, dstb, wb, rows0, rows1, zb, shared, sem0, sem1):
    cid = lax.axis_index("c")
    sid = lax.axis_index("s")
    cpt = src_hbm.shape[0] // NS
    rslice = npad // NS
    zrows = rslice // 5
    nvec = DH // 16

    def zf(i, carry):
        for k in range(nvec):
            zb[i, pl.ds(k * 16, 16)] = jnp.zeros((16,), jnp.float32)
        return carry
    lax.fori_loop(0, zrows, zf, 0)
    for j in range(5):
        pltpu.sync_copy(zb, shared.at[pl.ds(sid * rslice + j * zrows, zrows)])
    plsc.subcore_barrier()

    base = sid * cpt
    pltpu.sync_copy(src_hbm.at[pl.ds(base, cpt)], srcb)
    pltpu.sync_copy(dst_hbm.at[pl.ds(base, cpt)], dstb)
    pltpu.sync_copy(w_hbm.at[pl.ds(base, cpt)], wb)

    off = cid * n

    def obody(g, carry):
        for k in range(8):
            sl = pl.ds(k * 16, 16)
            srcb[g, sl] = srcb[g, sl] + off
        return carry
    lax.fori_loop(0, cpt, obody, 0)

    rows = (rows0, rows1)
    sems = (sem0, sem1)

    def start(c, b):
        pltpu.async_copy(u_hbm.at[srcb.at[c]], rows[b], sems[b])

    def wait(b):
        pltpu.make_async_copy(u_hbm.at[pl.ds(0, CHUNK)], rows[b], sems[b]).wait()

    start(0, 0)
    start(1, 1)

    def body(g, carry):
        for b in range(2):
            c = 2 * g + b
            wait(b)
            rb = rows[b]

            def ebody(j, ecarry):
                w16 = wb[c, pl.ds(j * 16, 16)]
                for t in range(16):
                    e = j * 16 + t
                    wv = w16[t]
                    for k in range(nvec):
                        sl = pl.ds(k * 16, 16)
                        rb[e, sl] = rb[e, sl] * wv
                return ecarry
            lax.fori_loop(0, CHUNK // 16, ebody, 0)
            pltpu.sync_copy(rb, shared.at[dstb.at[c]], add=True)

            @pl.when(c + 2 < cpt)
            def _():
                start(c + 2, b)
        return carry
    lax.fori_loop(0, cpt // 2, body, 0)
    plsc.subcore_barrier()
    for j in range(5):
        sl = pl.ds(sid * rslice + j * zrows, zrows)
        pltpu.sync_copy(shared.at[sl], out_hbm.at[cid, sl])


def _spmm_call(u2, src_p, dst_p, w_p, npad):
    n2, dh = u2.shape
    n = n2 // NC
    nchunks = src_p.shape[0]
    cpt = nchunks // NS
    return pl.kernel(
        functools.partial(_spmm_body, n, npad),
        out_type=jax.ShapeDtypeStruct((NC, npad, dh), jnp.float32),
        mesh=_MESH,
        scratch_types=[
            pltpu.VMEM((cpt, CHUNK), jnp.int32),
            pltpu.VMEM((cpt, CHUNK), jnp.int32),
            pltpu.VMEM((cpt, CHUNK), jnp.float32),
            pltpu.VMEM((CHUNK, dh), jnp.float32),
            pltpu.VMEM((CHUNK, dh), jnp.float32),
            pltpu.VMEM((npad // NS // 5, dh), jnp.float32),
            pltpu.VMEM_SHARED((npad, dh), jnp.float32),
            pltpu.SemaphoreType.DMA,
            pltpu.SemaphoreType.DMA,
        ],
        compiler_params=pltpu.CompilerParams(use_tc_tiling_on_sc=False),
    )(u2, src_p, dst_p, w_p)


def _bn_body(x_ref, g_ref, b_ref, out_ref):
    x = x_ref[...]
    m = jnp.mean(x, axis=0, keepdims=True)
    v = jnp.mean(x * x, axis=0, keepdims=True) - m * m
    out_ref[...] = (x - m) * lax.rsqrt(v + 1e-5) * g_ref[...] + b_ref[...]


def _split_u(u, out_ref):
    out_ref[0] = u[:, :DH]
    out_ref[1] = u[:, DH:]


def _lin1_body(x_ref, degp_ref, w_ref, out_u, out_dinv):
    deg = 1.0 + jnp.sum(degp_ref[...], axis=1, keepdims=True)
    dinv = lax.rsqrt(deg)
    out_dinv[...] = dinv
    u = jnp.dot(x_ref[...], w_ref[...],
                preferred_element_type=jnp.float32) * dinv
    _split_u(u, out_u)


def _merge(acc_ref, u_ref, n):
    acc = jnp.concatenate([acc_ref[0, :n, :], acc_ref[1, :n, :]], axis=1)
    u = jnp.concatenate([u_ref[0], u_ref[1]], axis=1)
    return acc + u


def _mid_body(acc_ref, u_ref, dinv_ref, b_ref, w_ref, out_u):
    n = u_ref.shape[1]
    dinv = dinv_ref[...]
    h = _merge(acc_ref, u_ref, n) * dinv + b_ref[...]
    h = jnp.maximum(h, 0.0)
    u = jnp.dot(h, w_ref[...], preferred_element_type=jnp.float32) * dinv
    _split_u(u, out_u)


def _fin_body(acc_ref, u_ref, dinv_ref, b_ref, out_ref):
    n = u_ref.shape[1]
    out_ref[...] = _merge(acc_ref, u_ref, n) * dinv_ref[...] + b_ref[...]


def kernel(x, edge_index, batchsize, edge_weight,
           W1, b1, W2, b2, W3, b3, gamma, beta):
    n, d = x.shape
    e = edge_weight.shape[0]
    f32 = jnp.float32

    src = edge_index[0].astype(jnp.int32)
    dst = edge_index[1].astype(jnp.int32)
    w = edge_weight.astype(f32)

    grp = 256 * CHUNK
    nch2 = -(-e // grp)
    e_pad = nch2 * grp
    pad = e_pad - e
    pad_idx = jnp.arange(pad, dtype=jnp.int32) % n
    src_p = jnp.concatenate([src, pad_idx]).reshape(e_pad // CHUNK, CHUNK)
    dst_p = jnp.concatenate([dst, pad_idx]).reshape(e_pad // CHUNK, CHUNK)
    w_p = jnp.concatenate([w, jnp.zeros((pad,), f32)]).reshape(
        e_pad // CHUNK, CHUNK)

    npad = -(-n // (NS * 16)) * (NS * 16)

    x_nor = pl.pallas_call(
        _bn_body, out_shape=jax.ShapeDtypeStruct((n, d), f32),
    )(x, gamma.reshape(1, d), beta.reshape(1, d))

    degp = _deg_call(dst_p, w_p, npad)
    degp_t = degp.T[:n]

    u1, dinv = pl.pallas_call(
        _lin1_body,
        out_shape=(jax.ShapeDtypeStruct((NC, n, DH), f32),
                   jax.ShapeDtypeStruct((n, 1), f32)),
    )(x, degp_t, W1)

    acc1 = _spmm_call(u1.reshape(NC * n, DH), src_p, dst_p, w_p, npad)
    u2 = pl.pallas_call(
        _mid_body, out_shape=jax.ShapeDtypeStruct((NC, n, DH), f32),
    )(acc1, u1, dinv, b1.reshape(1, d), W2)

    acc2 = _spmm_call(u2.reshape(NC * n, DH), src_p, dst_p, w_p, npad)
    u3 = pl.pallas_call(
        _mid_body, out_shape=jax.ShapeDtypeStruct((NC, n, DH), f32),
    )(acc2, u2, dinv, b2.reshape(1, d), W3)

    acc3 = _spmm_call(u3.reshape(NC * n, DH), src_p, dst_p, w_p, npad)
    h = pl.pallas_call(
        _fin_body, out_shape=jax.ShapeDtypeStruct((n, d), f32),
    )(acc3, u3, dinv, b3.reshape(1, d))

    return (h, x_nor)
